# Initial kernel scaffold; baseline (speedup 1.0000x reference)
import jax
import jax.numpy as jnp
from jax.experimental import pallas as pl


def kernel(hidden_states, gate_w, w_lin, w_v, w_1):
    raise NotImplementedError("write your pallas kernel here")


# speedup vs baseline: 1.4882x; 1.4882x over previous
import functools
import math

import jax
import jax.numpy as jnp
from jax import lax
from jax.experimental import pallas as pl
from jax.experimental.pallas import tpu as pltpu
from jax.experimental.pallas import tpu_sc as plsc

D = 1024
F = 2048
E = 8
K = 2
T = 2048
RT = T * K
BT = 256
FB = 512
NF = F // FB
NB = RT // BT + E
RPAD = NB * BT
NW = 32


def _router_body(x_ref, gw_ref, logits_ref, rw_ref, sel_ref):
    x = x_ref[...]
    logits = lax.dot_general(x, gw_ref[...], (((1,), (1,)), ((), ())),
                             preferred_element_type=jnp.float32)
    logits_ref[...] = logits
    m = jnp.max(logits, axis=1, keepdims=True)
    ex = jnp.exp(logits - m)
    p = ex / jnp.sum(ex, axis=1, keepdims=True)
    iota = lax.broadcasted_iota(jnp.int32, (T, E), 1)
    m1 = jnp.max(p, axis=1, keepdims=True)
    i1 = jnp.min(jnp.where(p == m1, iota, E), axis=1, keepdims=True)
    p2 = jnp.where(iota == i1, -1.0, p)
    m2 = jnp.max(p2, axis=1, keepdims=True)
    i2 = jnp.min(jnp.where(p2 == m2, iota, E), axis=1, keepdims=True)
    rw_ref[...] = jnp.concatenate([m1, m2], axis=1)
    sel_ref[...] = jnp.concatenate([i1, i2], axis=1)


def _router(x, gate_w):
    return pl.pallas_call(
        _router_body,
        out_shape=(
            jax.ShapeDtypeStruct((T, E), jnp.float32),
            jax.ShapeDtypeStruct((T, K), jnp.float32),
            jax.ShapeDtypeStruct((T, K), jnp.int32),
        ),
    )(x, gate_w)


def _sc_gather_body(x_hbm, idx_hbm, out_hbm, idx_v, rows_v, sem):
    wid = lax.axis_index("s") * 2 + lax.axis_index("c")
    rows_per = RPAD // NW
    ch = rows_per // 2
    base = wid * rows_per
    for c in range(2):
        off = base + c * ch
        pltpu.sync_copy(idx_hbm.at[pl.ds(off, ch)], idx_v)
        pltpu.async_copy(x_hbm.at[idx_v], rows_v, sem).wait()
        pltpu.sync_copy(rows_v, out_hbm.at[pl.ds(off, ch)])


def _sc_gather(x, sorted_tok):
    ch = RPAD // NW // 2
    mesh = plsc.VectorSubcoreMesh(core_axis_name="c", subcore_axis_name="s")
    k = functools.partial(
        pl.kernel,
        out_type=jax.ShapeDtypeStruct((RPAD, D), jnp.float32),
        mesh=mesh,
        scratch_types=[
            pltpu.VMEM((ch,), jnp.int32),
            pltpu.VMEM((ch, D), jnp.float32),
            pltpu.SemaphoreType.DMA,
        ],
    )(_sc_gather_body)
    return k(x, sorted_tok)


_INV_SQRT2 = 1.0 / math.sqrt(2.0)


def _mlp_body(be_ref, x_ref, wl_ref, wv_ref, w1_ref, w_ref, y_ref, acc_ref):
    f = pl.program_id(1)
    xb = x_ref[...]
    a = lax.dot_general(xb, wl_ref[0], (((1,), (1,)), ((), ())),
                        preferred_element_type=jnp.float32)
    v = lax.dot_general(xb, wv_ref[0], (((1,), (1,)), ((), ())),
                        preferred_element_type=jnp.float32)
    g = 0.5 * a * (1.0 + lax.erf(a * _INV_SQRT2))
    p = lax.dot_general(g * v, w1_ref[0], (((1,), (1,)), ((), ())),
                        preferred_element_type=jnp.float32)

    @pl.when(f == 0)
    def _():
        acc_ref[...] = p

    @pl.when(f > 0)
    def _():
        acc_ref[...] += p

    @pl.when(f == NF - 1)
    def _():
        y_ref[...] = acc_ref[...] * w_ref[...]


def _mlp(x_sorted, w_lin, w_v, w_1, sorted_w, block_expert):
    grid_spec = pltpu.PrefetchScalarGridSpec(
        num_scalar_prefetch=1,
        grid=(NB, NF),
        in_specs=[
            pl.BlockSpec((BT, D), lambda b, f, be: (b, 0)),
            pl.BlockSpec((1, FB, D), lambda b, f, be: (be[b], f, 0)),
            pl.BlockSpec((1, FB, D), lambda b, f, be: (be[b], f, 0)),
            pl.BlockSpec((1, D, FB), lambda b, f, be: (be[b], 0, f)),
            pl.BlockSpec((BT, 1), lambda b, f, be: (b, 0)),
        ],
        out_specs=pl.BlockSpec((BT, D), lambda b, f, be: (b, 0)),
        scratch_shapes=[pltpu.VMEM((BT, D), jnp.float32)],
    )
    return pl.pallas_call(
        _mlp_body,
        grid_spec=grid_spec,
        out_shape=jax.ShapeDtypeStruct((RPAD, D), jnp.float32),
    )(block_expert, x_sorted, w_lin, w_v, w_1, sorted_w)


def _sc_combine_body(y_hbm, pos0_hbm, pos1_hbm, out_hbm,
                     i0, i1, r0, r1, sem0, sem1):
    wid = lax.axis_index("s") * 2 + lax.axis_index("c")
    tokens_per = T // NW
    tch = tokens_per // 2
    for c in range(2):
        tbase = wid * tokens_per + c * tch
        pltpu.sync_copy(pos0_hbm.at[pl.ds(tbase, tch)], i0)
        pltpu.sync_copy(pos1_hbm.at[pl.ds(tbase, tch)], i1)
        d0 = pltpu.async_copy(y_hbm.at[i0], r0, sem0)
        d1 = pltpu.async_copy(y_hbm.at[i1], r1, sem1)
        d0.wait()
        d1.wait()

        def row_add(r, carry):
            for j in range(D // 16):
                sl = pl.ds(j * 16, 16)
                r0[r, sl] += r1[r, sl]
            return carry

        lax.fori_loop(0, tch, row_add, 0)
        pltpu.sync_copy(r0, out_hbm.at[pl.ds(tbase, tch)])


def _sc_combine(y_rows, pos0, pos1):
    tch = T // NW // 2
    mesh = plsc.VectorSubcoreMesh(core_axis_name="c", subcore_axis_name="s")
    k = functools.partial(
        pl.kernel,
        out_type=jax.ShapeDtypeStruct((T, D), jnp.float32),
        mesh=mesh,
        scratch_types=[
            pltpu.VMEM((tch,), jnp.int32),
            pltpu.VMEM((tch,), jnp.int32),
            pltpu.VMEM((tch, D), jnp.float32),
            pltpu.VMEM((tch, D), jnp.float32),
            pltpu.SemaphoreType.DMA,
            pltpu.SemaphoreType.DMA,
        ],
    )(_sc_combine_body)
    return k(y_rows, pos0, pos1)


def kernel(hidden_states, gate_w, w_lin, w_v, w_1):
    x = hidden_states.reshape(T, D)
    logits, rw, sel = _router(x, gate_w)

    flat_e = sel.reshape(-1)
    oh = (flat_e[:, None] == jnp.arange(E, dtype=jnp.int32)[None, :])
    oh = oh.astype(jnp.int32)
    counts = jnp.sum(oh, axis=0)
    ranks = jnp.take_along_axis(jnp.cumsum(oh, axis=0) - oh,
                                flat_e[:, None], axis=1)[:, 0]
    nblk = (counts + BT - 1) // BT
    blk_end = jnp.cumsum(nblk)
    padded_start = (blk_end - nblk) * BT
    pos = padded_start[flat_e] + ranks
    sorted_pair = jnp.zeros((RPAD,), jnp.int32).at[pos].set(
        jnp.arange(RT, dtype=jnp.int32))
    sorted_tok = sorted_pair // K
    sorted_w = jnp.zeros((RPAD,), jnp.float32).at[pos].set(
        rw.reshape(-1)).reshape(RPAD, 1)
    block_expert = jnp.searchsorted(
        blk_end, jnp.arange(NB, dtype=jnp.int32), side="right")
    block_expert = jnp.minimum(block_expert, E - 1).astype(jnp.int32)
    pos_r = pos.reshape(T, K)
    pos0 = pos_r[:, 0].astype(jnp.int32)
    pos1 = pos_r[:, 1].astype(jnp.int32)

    x_sorted = _sc_gather(x, sorted_tok)
    y_rows = _mlp(x_sorted, w_lin, w_v, w_1, sorted_w, block_expert)
    final = _sc_combine(y_rows, pos0, pos1)
    return final.reshape(1, T, D), logits


# speedup vs baseline: 1.5371x; 1.0329x over previous
import functools
import math

import jax
import jax.numpy as jnp
from jax import lax
from jax.experimental import pallas as pl
from jax.experimental.pallas import tpu as pltpu
from jax.experimental.pallas import tpu_sc as plsc

D = 1024
F = 2048
E = 8
K = 2
T = 2048
RT = T * K
BT = 256
FB = 512
NF = F // FB
NB = RT // BT + E
RPAD = NB * BT
NW = 32


def _router_body(x_ref, gw_ref, logits_ref, rw_ref, sel_ref):
    x = x_ref[...]
    logits = lax.dot_general(x, gw_ref[...], (((1,), (1,)), ((), ())),
                             preferred_element_type=jnp.float32)
    logits_ref[...] = logits
    m = jnp.max(logits, axis=1, keepdims=True)
    ex = jnp.exp(logits - m)
    p = ex / jnp.sum(ex, axis=1, keepdims=True)
    iota = lax.broadcasted_iota(jnp.int32, (T, E), 1)
    m1 = jnp.max(p, axis=1, keepdims=True)
    i1 = jnp.min(jnp.where(p == m1, iota, E), axis=1, keepdims=True)
    p2 = jnp.where(iota == i1, -1.0, p)
    m2 = jnp.max(p2, axis=1, keepdims=True)
    i2 = jnp.min(jnp.where(p2 == m2, iota, E), axis=1, keepdims=True)
    rw_ref[...] = jnp.concatenate([m1, m2], axis=1)
    sel_ref[...] = jnp.concatenate([i1, i2], axis=1)


def _router(x, gate_w):
    return pl.pallas_call(
        _router_body,
        out_shape=(
            jax.ShapeDtypeStruct((T, E), jnp.float32),
            jax.ShapeDtypeStruct((T, K), jnp.float32),
            jax.ShapeDtypeStruct((T, K), jnp.int32),
        ),
    )(x, gate_w)


def _sc_gather_body(x_hbm, idx_hbm, out_hbm, idx_v, rows_v, sem):
    wid = lax.axis_index("s") * 2 + lax.axis_index("c")
    rows_per = RPAD // NW
    ch = rows_per // 2
    base = wid * rows_per
    for c in range(2):
        off = base + c * ch
        pltpu.sync_copy(idx_hbm.at[pl.ds(off, ch)], idx_v)
        pltpu.async_copy(x_hbm.at[idx_v], rows_v, sem).wait()
        pltpu.sync_copy(rows_v, out_hbm.at[pl.ds(off, ch)])


def _sc_gather(x, sorted_tok):
    ch = RPAD // NW // 2
    mesh = plsc.VectorSubcoreMesh(core_axis_name="c", subcore_axis_name="s")
    k = functools.partial(
        pl.kernel,
        out_type=jax.ShapeDtypeStruct((RPAD, D), jnp.float32),
        mesh=mesh,
        scratch_types=[
            pltpu.VMEM((ch,), jnp.int32),
            pltpu.VMEM((ch, D), jnp.float32),
            pltpu.SemaphoreType.DMA,
        ],
    )(_sc_gather_body)
    return k(x, sorted_tok)


_INV_SQRT2 = 1.0 / math.sqrt(2.0)


def _mlp_body(be_ref, x_ref, wl_ref, wv_ref, w1_ref, w_ref, y_ref):
    f = pl.program_id(0)
    b = pl.program_id(1)
    xb = x_ref[...]
    a = lax.dot_general(xb, wl_ref[0], (((1,), (1,)), ((), ())),
                        preferred_element_type=jnp.float32)
    v = lax.dot_general(xb, wv_ref[0], (((1,), (1,)), ((), ())),
                        preferred_element_type=jnp.float32)
    g = 0.5 * a * (1.0 + lax.erf(a * _INV_SQRT2))
    p = lax.dot_general(g * v, w1_ref[0], (((1,), (1,)), ((), ())),
                        preferred_element_type=jnp.float32)
    row = pl.ds(b * BT, BT)

    @pl.when(f == 0)
    def _():
        y_ref[row, :] = p

    @pl.when(jnp.logical_and(f > 0, f < NF - 1))
    def _():
        y_ref[row, :] += p

    @pl.when(f == NF - 1)
    def _():
        y_ref[row, :] = (y_ref[row, :] + p) * w_ref[...]


def _mlp(x_sorted, w_lin, w_v, w_1, sorted_w, block_expert):
    grid_spec = pltpu.PrefetchScalarGridSpec(
        num_scalar_prefetch=1,
        grid=(NF, NB),
        in_specs=[
            pl.BlockSpec((BT, D), lambda f, b, be: (b, 0)),
            pl.BlockSpec((1, FB, D), lambda f, b, be: (be[b], f, 0)),
            pl.BlockSpec((1, FB, D), lambda f, b, be: (be[b], f, 0)),
            pl.BlockSpec((1, D, FB), lambda f, b, be: (be[b], 0, f)),
            pl.BlockSpec((BT, 1), lambda f, b, be: (b, 0)),
        ],
        out_specs=pl.BlockSpec((RPAD, D), lambda f, b, be: (0, 0)),
    )
    return pl.pallas_call(
        _mlp_body,
        grid_spec=grid_spec,
        out_shape=jax.ShapeDtypeStruct((RPAD, D), jnp.float32),
    )(block_expert, x_sorted, w_lin, w_v, w_1, sorted_w)


def _sc_combine_body(y_hbm, pos0_hbm, pos1_hbm, out_hbm,
                     i0, i1, r0, r1, rout, sem0, sem1):
    wid = lax.axis_index("s") * 2 + lax.axis_index("c")
    tokens_per = T // NW
    tch = tokens_per // 2
    for c in range(2):
        tbase = wid * tokens_per + c * tch
        pltpu.sync_copy(pos0_hbm.at[pl.ds(tbase, tch)], i0)
        pltpu.sync_copy(pos1_hbm.at[pl.ds(tbase, tch)], i1)
        d0 = pltpu.async_copy(y_hbm.at[i0], r0, sem0)
        d1 = pltpu.async_copy(y_hbm.at[i1], r1, sem1)
        d0.wait()
        d1.wait()

        @plsc.parallel_loop(0, tch, 1, unroll=2)
        def _(r):
            for j in range(D // 16):
                sl = pl.ds(j * 16, 16)
                rout[r, sl] = r0[r, sl] + r1[r, sl]

        pltpu.sync_copy(rout, out_hbm.at[pl.ds(tbase, tch)])


def _sc_combine(y_rows, pos0, pos1):
    tch = T // NW // 2
    mesh = plsc.VectorSubcoreMesh(core_axis_name="c", subcore_axis_name="s")
    k = functools.partial(
        pl.kernel,
        out_type=jax.ShapeDtypeStruct((T, D), jnp.float32),
        mesh=mesh,
        scratch_types=[
            pltpu.VMEM((tch,), jnp.int32),
            pltpu.VMEM((tch,), jnp.int32),
            pltpu.VMEM((tch, D), jnp.float32),
            pltpu.VMEM((tch, D), jnp.float32),
            pltpu.VMEM((tch, D), jnp.float32),
            pltpu.SemaphoreType.DMA,
            pltpu.SemaphoreType.DMA,
        ],
    )(_sc_combine_body)
    return k(y_rows, pos0, pos1)


def kernel(hidden_states, gate_w, w_lin, w_v, w_1):
    x = hidden_states.reshape(T, D)
    logits, rw, sel = _router(x, gate_w)

    flat_e = sel.reshape(-1)
    oh = (flat_e[:, None] == jnp.arange(E, dtype=jnp.int32)[None, :])
    oh = oh.astype(jnp.int32)
    counts = jnp.sum(oh, axis=0)
    ranks = jnp.take_along_axis(jnp.cumsum(oh, axis=0) - oh,
                                flat_e[:, None], axis=1)[:, 0]
    nblk = (counts + BT - 1) // BT
    blk_end = jnp.cumsum(nblk)
    padded_start = (blk_end - nblk) * BT
    pos = padded_start[flat_e] + ranks
    sorted_pair = jnp.zeros((RPAD,), jnp.int32).at[pos].set(
        jnp.arange(RT, dtype=jnp.int32))
    sorted_tok = sorted_pair // K
    sorted_w = jnp.zeros((RPAD,), jnp.float32).at[pos].set(
        rw.reshape(-1)).reshape(RPAD, 1)
    block_expert = jnp.searchsorted(
        blk_end, jnp.arange(NB, dtype=jnp.int32), side="right")
    block_expert = jnp.minimum(block_expert, E - 1).astype(jnp.int32)
    pos_r = pos.reshape(T, K)
    pos0 = pos_r[:, 0].astype(jnp.int32)
    pos1 = pos_r[:, 1].astype(jnp.int32)

    x_sorted = _sc_gather(x, sorted_tok)
    y_rows = _mlp(x_sorted, w_lin, w_v, w_1, sorted_w, block_expert)
    final = _sc_combine(y_rows, pos0, pos1)
    return final.reshape(1, T, D), logits


# speedup vs baseline: 1.5378x; 1.0005x over previous
import functools
import math

import jax
import jax.numpy as jnp
from jax import lax
from jax.experimental import pallas as pl
from jax.experimental.pallas import tpu as pltpu
from jax.experimental.pallas import tpu_sc as plsc

D = 1024
F = 2048
E = 8
K = 2
T = 2048
RT = T * K
BT = 256
FB = 512
NF = F // FB
NB = RT // BT + E
RPAD = NB * BT
NW = 32


def _router_body(x_ref, gw_ref, logits_ref, rw_ref, sel_ref):
    x = x_ref[...]
    logits = lax.dot_general(x, gw_ref[...], (((1,), (1,)), ((), ())),
                             preferred_element_type=jnp.float32)
    logits_ref[...] = logits
    m = jnp.max(logits, axis=1, keepdims=True)
    ex = jnp.exp(logits - m)
    p = ex / jnp.sum(ex, axis=1, keepdims=True)
    iota = lax.broadcasted_iota(jnp.int32, (T, E), 1)
    m1 = jnp.max(p, axis=1, keepdims=True)
    i1 = jnp.min(jnp.where(p == m1, iota, E), axis=1, keepdims=True)
    p2 = jnp.where(iota == i1, -1.0, p)
    m2 = jnp.max(p2, axis=1, keepdims=True)
    i2 = jnp.min(jnp.where(p2 == m2, iota, E), axis=1, keepdims=True)
    rw_ref[...] = jnp.concatenate([m1, m2], axis=1)
    sel_ref[...] = jnp.concatenate([i1, i2], axis=1)


def _router(x, gate_w):
    return pl.pallas_call(
        _router_body,
        out_shape=(
            jax.ShapeDtypeStruct((T, E), jnp.float32),
            jax.ShapeDtypeStruct((T, K), jnp.float32),
            jax.ShapeDtypeStruct((T, K), jnp.int32),
        ),
    )(x, gate_w)


_GCH = RPAD // NW // 4


def _sc_gather_body(x_hbm, idx_hbm, out_hbm, idx_v, b0, b1,
                    gs0, gs1, ws0, ws1):
    wid = lax.axis_index("s") * 2 + lax.axis_index("c")
    rows_per = RPAD // NW
    base = wid * rows_per
    pltpu.sync_copy(idx_hbm.at[pl.ds(base, rows_per)], idx_v)
    bufs = (b0, b1)
    gsems = (gs0, gs1)
    wsems = (ws0, ws1)

    def gather(c):
        return pltpu.async_copy(
            x_hbm.at[idx_v.at[pl.ds(c * _GCH, _GCH)]], bufs[c % 2],
            gsems[c % 2])

    def writeback(c):
        return pltpu.async_copy(
            bufs[c % 2], out_hbm.at[pl.ds(base + c * _GCH, _GCH)],
            wsems[c % 2])

    g0 = gather(0)
    g1 = gather(1)
    g0.wait()
    w0 = writeback(0)
    g1.wait()
    w1 = writeback(1)
    w0.wait()
    g2 = gather(2)
    w1.wait()
    g3 = gather(3)
    g2.wait()
    w2 = writeback(2)
    g3.wait()
    w3 = writeback(3)
    w2.wait()
    w3.wait()


def _sc_gather(x, sorted_tok):
    mesh = plsc.VectorSubcoreMesh(core_axis_name="c", subcore_axis_name="s")
    k = functools.partial(
        pl.kernel,
        out_type=jax.ShapeDtypeStruct((RPAD, D), jnp.float32),
        mesh=mesh,
        scratch_types=[
            pltpu.VMEM((RPAD // NW,), jnp.int32),
            pltpu.VMEM((_GCH, D), jnp.float32),
            pltpu.VMEM((_GCH, D), jnp.float32),
            pltpu.SemaphoreType.DMA,
            pltpu.SemaphoreType.DMA,
            pltpu.SemaphoreType.DMA,
            pltpu.SemaphoreType.DMA,
        ],
    )(_sc_gather_body)
    return k(x, sorted_tok)


_INV_SQRT2 = 1.0 / math.sqrt(2.0)


def _mlp_body(be_ref, x_ref, wl_ref, wv_ref, w1_ref, w_ref, y_ref):
    f = pl.program_id(0)
    b = pl.program_id(1)
    xb = x_ref[...]
    a = lax.dot_general(xb, wl_ref[0], (((1,), (1,)), ((), ())),
                        preferred_element_type=jnp.float32)
    v = lax.dot_general(xb, wv_ref[0], (((1,), (1,)), ((), ())),
                        preferred_element_type=jnp.float32)
    g = 0.5 * a * (1.0 + lax.erf(a * _INV_SQRT2))
    p = lax.dot_general(g * v, w1_ref[0], (((1,), (1,)), ((), ())),
                        preferred_element_type=jnp.float32)
    row = pl.ds(b * BT, BT)

    @pl.when(f == 0)
    def _():
        y_ref[row, :] = p

    @pl.when(jnp.logical_and(f > 0, f < NF - 1))
    def _():
        y_ref[row, :] += p

    @pl.when(f == NF - 1)
    def _():
        y_ref[row, :] = (y_ref[row, :] + p) * w_ref[...]


def _mlp(x_sorted, w_lin, w_v, w_1, sorted_w, block_expert):
    grid_spec = pltpu.PrefetchScalarGridSpec(
        num_scalar_prefetch=1,
        grid=(NF, NB),
        in_specs=[
            pl.BlockSpec((BT, D), lambda f, b, be: (b, 0)),
            pl.BlockSpec((1, FB, D), lambda f, b, be: (be[b], f, 0)),
            pl.BlockSpec((1, FB, D), lambda f, b, be: (be[b], f, 0)),
            pl.BlockSpec((1, D, FB), lambda f, b, be: (be[b], 0, f)),
            pl.BlockSpec((BT, 1), lambda f, b, be: (b, 0)),
        ],
        out_specs=pl.BlockSpec((RPAD, D), lambda f, b, be: (0, 0)),
    )
    return pl.pallas_call(
        _mlp_body,
        grid_spec=grid_spec,
        out_shape=jax.ShapeDtypeStruct((RPAD, D), jnp.float32),
    )(block_expert, x_sorted, w_lin, w_v, w_1, sorted_w)


def _sc_combine_body(y_hbm, pos0_hbm, pos1_hbm, out_hbm,
                     i0, i1, r0, r1, rout, sem0, sem1):
    wid = lax.axis_index("s") * 2 + lax.axis_index("c")
    tokens_per = T // NW
    tch = tokens_per // 2
    for c in range(2):
        tbase = wid * tokens_per + c * tch
        pltpu.sync_copy(pos0_hbm.at[pl.ds(tbase, tch)], i0)
        pltpu.sync_copy(pos1_hbm.at[pl.ds(tbase, tch)], i1)
        d0 = pltpu.async_copy(y_hbm.at[i0], r0, sem0)
        d1 = pltpu.async_copy(y_hbm.at[i1], r1, sem1)
        d0.wait()
        d1.wait()

        @plsc.parallel_loop(0, tch, 1, unroll=2)
        def _(r):
            for j in range(D // 16):
                sl = pl.ds(j * 16, 16)
                rout[r, sl] = r0[r, sl] + r1[r, sl]

        pltpu.sync_copy(rout, out_hbm.at[pl.ds(tbase, tch)])


def _sc_combine(y_rows, pos0, pos1):
    tch = T // NW // 2
    mesh = plsc.VectorSubcoreMesh(core_axis_name="c", subcore_axis_name="s")
    k = functools.partial(
        pl.kernel,
        out_type=jax.ShapeDtypeStruct((T, D), jnp.float32),
        mesh=mesh,
        scratch_types=[
            pltpu.VMEM((tch,), jnp.int32),
            pltpu.VMEM((tch,), jnp.int32),
            pltpu.VMEM((tch, D), jnp.float32),
            pltpu.VMEM((tch, D), jnp.float32),
            pltpu.VMEM((tch, D), jnp.float32),
            pltpu.SemaphoreType.DMA,
            pltpu.SemaphoreType.DMA,
        ],
    )(_sc_combine_body)
    return k(y_rows, pos0, pos1)


def kernel(hidden_states, gate_w, w_lin, w_v, w_1):
    x = hidden_states.reshape(T, D)
    logits, rw, sel = _router(x, gate_w)

    flat_e = sel.reshape(-1)
    oh = (flat_e[:, None] == jnp.arange(E, dtype=jnp.int32)[None, :])
    oh = oh.astype(jnp.int32)
    counts = jnp.sum(oh, axis=0)
    ranks = jnp.take_along_axis(jnp.cumsum(oh, axis=0) - oh,
                                flat_e[:, None], axis=1)[:, 0]
    nblk = (counts + BT - 1) // BT
    blk_end = jnp.cumsum(nblk)
    padded_start = (blk_end - nblk) * BT
    pos = padded_start[flat_e] + ranks
    sorted_pair = jnp.zeros((RPAD,), jnp.int32).at[pos].set(
        jnp.arange(RT, dtype=jnp.int32))
    sorted_tok = sorted_pair // K
    sorted_w = jnp.zeros((RPAD,), jnp.float32).at[pos].set(
        rw.reshape(-1)).reshape(RPAD, 1)
    block_expert = jnp.searchsorted(
        blk_end, jnp.arange(NB, dtype=jnp.int32), side="right")
    block_expert = jnp.minimum(block_expert, E - 1).astype(jnp.int32)
    pos_r = pos.reshape(T, K)
    pos0 = pos_r[:, 0].astype(jnp.int32)
    pos1 = pos_r[:, 1].astype(jnp.int32)

    x_sorted = _sc_gather(x, sorted_tok)
    y_rows = _mlp(x_sorted, w_lin, w_v, w_1, sorted_w, block_expert)
    final = _sc_combine(y_rows, pos0, pos1)
    return final.reshape(1, T, D), logits


# speedup vs baseline: 1.5391x; 1.0009x over previous
import functools
import math

import jax
import jax.numpy as jnp
from jax import lax
from jax.experimental import pallas as pl
from jax.experimental.pallas import tpu as pltpu
from jax.experimental.pallas import tpu_sc as plsc

D = 1024
F = 2048
E = 8
K = 2
T = 2048
RT = T * K
BT = 256
FB = 512
NF = F // FB
NB = RT // BT + E
RPAD = NB * BT
NW = 32


def _router_body(x_ref, gw_ref, logits_ref, rw_ref, sel_ref):
    x = x_ref[...]
    logits = lax.dot_general(x, gw_ref[...], (((1,), (1,)), ((), ())),
                             preferred_element_type=jnp.float32)
    logits_ref[...] = logits
    m = jnp.max(logits, axis=1, keepdims=True)
    ex = jnp.exp(logits - m)
    p = ex / jnp.sum(ex, axis=1, keepdims=True)
    iota = lax.broadcasted_iota(jnp.int32, (T, E), 1)
    m1 = jnp.max(p, axis=1, keepdims=True)
    i1 = jnp.min(jnp.where(p == m1, iota, E), axis=1, keepdims=True)
    p2 = jnp.where(iota == i1, -1.0, p)
    m2 = jnp.max(p2, axis=1, keepdims=True)
    i2 = jnp.min(jnp.where(p2 == m2, iota, E), axis=1, keepdims=True)
    rw_ref[...] = jnp.concatenate([m1, m2], axis=1)
    sel_ref[...] = jnp.concatenate([i1, i2], axis=1)


def _router(x, gate_w):
    return pl.pallas_call(
        _router_body,
        out_shape=(
            jax.ShapeDtypeStruct((T, E), jnp.float32),
            jax.ShapeDtypeStruct((T, K), jnp.float32),
            jax.ShapeDtypeStruct((T, K), jnp.int32),
        ),
    )(x, gate_w)


def _sc_gather_body(x_hbm, idx_hbm, out_hbm, idx_v, rows_v, sem):
    wid = lax.axis_index("s") * 2 + lax.axis_index("c")
    rows_per = idx_v.shape[0]
    base = wid * rows_per
    pltpu.sync_copy(idx_hbm.at[pl.ds(base, rows_per)], idx_v)
    pltpu.async_copy(x_hbm.at[idx_v], rows_v, sem).wait()
    pltpu.sync_copy(rows_v, out_hbm.at[pl.ds(base, rows_per)])


def _sc_gather(x, sorted_tok):
    n = sorted_tok.shape[0]
    rows_per = n // NW
    mesh = plsc.VectorSubcoreMesh(core_axis_name="c", subcore_axis_name="s")
    k = functools.partial(
        pl.kernel,
        out_type=jax.ShapeDtypeStruct((n, D), jnp.float32),
        mesh=mesh,
        scratch_types=[
            pltpu.VMEM((rows_per,), jnp.int32),
            pltpu.VMEM((rows_per, D), jnp.float32),
            pltpu.SemaphoreType.DMA,
        ],
    )(_sc_gather_body)
    return k(x, sorted_tok)


_INV_SQRT2 = 1.0 / math.sqrt(2.0)


def _mlp_body(be_ref, x_ref, wl_ref, wv_ref, w1_ref, w_ref, y_ref):
    f = pl.program_id(0)
    b = pl.program_id(1)
    xb = x_ref[...]
    a = lax.dot_general(xb, wl_ref[0], (((1,), (1,)), ((), ())),
                        preferred_element_type=jnp.float32)
    v = lax.dot_general(xb, wv_ref[0], (((1,), (1,)), ((), ())),
                        preferred_element_type=jnp.float32)
    g = 0.5 * a * (1.0 + lax.erf(a * _INV_SQRT2))
    p = lax.dot_general(g * v, w1_ref[0], (((1,), (1,)), ((), ())),
                        preferred_element_type=jnp.float32)
    row = pl.ds(b * BT, BT)

    @pl.when(f == 0)
    def _():
        y_ref[row, :] = p

    @pl.when(jnp.logical_and(f > 0, f < NF - 1))
    def _():
        y_ref[row, :] += p

    @pl.when(f == NF - 1)
    def _():
        y_ref[row, :] = (y_ref[row, :] + p) * w_ref[...]


def _mlp(x_sorted, w_lin, w_v, w_1, sorted_w, block_expert):
    nrows = x_sorted.shape[0]
    nb = nrows // BT
    grid_spec = pltpu.PrefetchScalarGridSpec(
        num_scalar_prefetch=1,
        grid=(NF, nb),
        in_specs=[
            pl.BlockSpec((BT, D), lambda f, b, be: (b, 0)),
            pl.BlockSpec((1, FB, D), lambda f, b, be: (be[b], f, 0)),
            pl.BlockSpec((1, FB, D), lambda f, b, be: (be[b], f, 0)),
            pl.BlockSpec((1, D, FB), lambda f, b, be: (be[b], 0, f)),
            pl.BlockSpec((BT, 1), lambda f, b, be: (b, 0)),
        ],
        out_specs=pl.BlockSpec((nrows, D), lambda f, b, be: (0, 0)),
    )
    return pl.pallas_call(
        _mlp_body,
        grid_spec=grid_spec,
        out_shape=jax.ShapeDtypeStruct((nrows, D), jnp.float32),
    )(block_expert, x_sorted, w_lin, w_v, w_1, sorted_w)


def _sc_combine_body(y_hbm, pos0_hbm, pos1_hbm, out_hbm,
                     i0, i1, r0, r1, rout, sem0, sem1):
    wid = lax.axis_index("s") * 2 + lax.axis_index("c")
    tokens_per = T // NW
    tch = tokens_per // 2
    for c in range(2):
        tbase = wid * tokens_per + c * tch
        pltpu.sync_copy(pos0_hbm.at[pl.ds(tbase, tch)], i0)
        pltpu.sync_copy(pos1_hbm.at[pl.ds(tbase, tch)], i1)
        d0 = pltpu.async_copy(y_hbm.at[i0], r0, sem0)
        d1 = pltpu.async_copy(y_hbm.at[i1], r1, sem1)
        d0.wait()
        d1.wait()

        @plsc.parallel_loop(0, tch, 1, unroll=2)
        def _(r):
            for j in range(D // 16):
                sl = pl.ds(j * 16, 16)
                rout[r, sl] = r0[r, sl] + r1[r, sl]

        pltpu.sync_copy(rout, out_hbm.at[pl.ds(tbase, tch)])


def _sc_combine(y_rows, pos0, pos1):
    tch = T // NW // 2
    mesh = plsc.VectorSubcoreMesh(core_axis_name="c", subcore_axis_name="s")
    k = functools.partial(
        pl.kernel,
        out_type=jax.ShapeDtypeStruct((T, D), jnp.float32),
        mesh=mesh,
        scratch_types=[
            pltpu.VMEM((tch,), jnp.int32),
            pltpu.VMEM((tch,), jnp.int32),
            pltpu.VMEM((tch, D), jnp.float32),
            pltpu.VMEM((tch, D), jnp.float32),
            pltpu.VMEM((tch, D), jnp.float32),
            pltpu.SemaphoreType.DMA,
            pltpu.SemaphoreType.DMA,
        ],
    )(_sc_combine_body)
    return k(y_rows, pos0, pos1)


def kernel(hidden_states, gate_w, w_lin, w_v, w_1):
    x = hidden_states.reshape(T, D)
    logits, rw, sel = _router(x, gate_w)

    flat_e = sel.reshape(-1)
    oh = (flat_e[:, None] == jnp.arange(E, dtype=jnp.int32)[None, :])
    oh = oh.astype(jnp.int32)
    counts = jnp.sum(oh, axis=0)
    ranks = jnp.sum((jnp.cumsum(oh, axis=0) - oh) * oh, axis=1)
    nblk = (counts + BT - 1) // BT
    blk_end = jnp.cumsum(nblk)
    padded_start = (blk_end - nblk) * BT
    pos = jnp.sum(padded_start[None, :] * oh, axis=1) + ranks
    sorted_pair = jnp.zeros((RPAD,), jnp.int32).at[pos].set(
        jnp.arange(RT, dtype=jnp.int32))
    sorted_tok = sorted_pair // K
    sorted_w = jnp.zeros((RPAD,), jnp.float32).at[pos].set(
        rw.reshape(-1)).reshape(RPAD, 1)
    block_expert = jnp.searchsorted(
        blk_end, jnp.arange(NB, dtype=jnp.int32), side="right")
    block_expert = jnp.minimum(block_expert, E - 1).astype(jnp.int32)
    pos_r = pos.reshape(T, K)
    pos0 = pos_r[:, 0].astype(jnp.int32)
    pos1 = pos_r[:, 1].astype(jnp.int32)

    half = RPAD // 2
    hb = NB // 2
    x_a = _sc_gather(x, sorted_tok[:half])
    x_b = _sc_gather(x, sorted_tok[half:])
    y_a = _mlp(x_a, w_lin, w_v, w_1, sorted_w[:half], block_expert[:hb])
    y_b = _mlp(x_b, w_lin, w_v, w_1, sorted_w[half:], block_expert[hb:])
    y_rows = jnp.concatenate([y_a, y_b], axis=0)
    final = _sc_combine(y_rows, pos0, pos1)
    return final.reshape(1, T, D), logits


# speedup vs baseline: 2.3166x; 1.5052x over previous
import functools
import math

import jax
import jax.numpy as jnp
from jax import lax
from jax.experimental import pallas as pl
from jax.experimental.pallas import tpu as pltpu
from jax.experimental.pallas import tpu_sc as plsc

D = 1024
F = 2048
E = 8
K = 2
T = 2048
RT = T * K
BT = 256
FB = 512
NF = F // FB
NB = RT // BT + E
RPAD = NB * BT
NW = 32


def _router_body(x_ref, gw_ref, logits_ref, rw_ref, sel_ref):
    x = x_ref[...]
    logits = lax.dot_general(x, gw_ref[...], (((1,), (1,)), ((), ())),
                             preferred_element_type=jnp.float32)
    logits_ref[...] = logits
    m = jnp.max(logits, axis=1, keepdims=True)
    ex = jnp.exp(logits - m)
    p = ex / jnp.sum(ex, axis=1, keepdims=True)
    iota = lax.broadcasted_iota(jnp.int32, (T, E), 1)
    m1 = jnp.max(p, axis=1, keepdims=True)
    i1 = jnp.min(jnp.where(p == m1, iota, E), axis=1, keepdims=True)
    p2 = jnp.where(iota == i1, -1.0, p)
    m2 = jnp.max(p2, axis=1, keepdims=True)
    i2 = jnp.min(jnp.where(p2 == m2, iota, E), axis=1, keepdims=True)
    rw_ref[...] = jnp.concatenate([m1, m2], axis=1)
    sel_ref[...] = jnp.concatenate([i1, i2], axis=1)


def _router(x, gate_w):
    return pl.pallas_call(
        _router_body,
        out_shape=(
            jax.ShapeDtypeStruct((T, E), jnp.float32),
            jax.ShapeDtypeStruct((T, K), jnp.float32),
            jax.ShapeDtypeStruct((T, K), jnp.int32),
        ),
    )(x, gate_w)


_TPW = T // NW


def _sc_dispatch_body(x_hbm, posb_hbm, out_hbm, xbuf, i0, i1, sem0, sem1):
    wid = lax.axis_index("s") * 2 + lax.axis_index("c")
    pltpu.sync_copy(x_hbm.at[pl.ds(wid * _TPW, _TPW)], xbuf)
    pltpu.sync_copy(posb_hbm.at[wid, 0], i0)
    pltpu.sync_copy(posb_hbm.at[wid, 1], i1)
    d0 = pltpu.async_copy(xbuf, out_hbm.at[i0], sem0)
    d1 = pltpu.async_copy(xbuf, out_hbm.at[i1], sem1)
    d0.wait()
    d1.wait()


def _sc_dispatch(x, posb):
    mesh = plsc.VectorSubcoreMesh(core_axis_name="c", subcore_axis_name="s")
    k = functools.partial(
        pl.kernel,
        out_type=jax.ShapeDtypeStruct((RPAD, D), jnp.float32),
        mesh=mesh,
        scratch_types=[
            pltpu.VMEM((_TPW, D), jnp.float32),
            pltpu.VMEM((_TPW,), jnp.int32),
            pltpu.VMEM((_TPW,), jnp.int32),
            pltpu.SemaphoreType.DMA,
            pltpu.SemaphoreType.DMA,
        ],
    )(_sc_dispatch_body)
    return k(x, posb)


_INV_SQRT2 = 1.0 / math.sqrt(2.0)


def _mlp_body(be_ref, nr_ref, x_ref, wl_ref, wv_ref, w1_ref, w_ref, y_ref):
    f = pl.program_id(0)
    b = pl.program_id(1)

    @pl.when(nr_ref[b] > 0)
    def _():
        xb = x_ref[...]
        a = lax.dot_general(xb, wl_ref[0], (((1,), (1,)), ((), ())),
                            preferred_element_type=jnp.float32)
        v = lax.dot_general(xb, wv_ref[0], (((1,), (1,)), ((), ())),
                            preferred_element_type=jnp.float32)
        g = 0.5 * a * (1.0 + lax.erf(a * _INV_SQRT2))
        p = lax.dot_general(g * v, w1_ref[0], (((1,), (1,)), ((), ())),
                            preferred_element_type=jnp.float32)
        row = pl.ds(b * BT, BT)

        @pl.when(f == 0)
        def _():
            y_ref[row, :] = p

        @pl.when(jnp.logical_and(f > 0, f < NF - 1))
        def _():
            y_ref[row, :] += p

        @pl.when(f == NF - 1)
        def _():
            y_ref[row, :] = (y_ref[row, :] + p) * w_ref[...]


def _mlp(x_sorted, w_lin, w_v, w_1, sorted_w, block_expert, block_nrows):
    grid_spec = pltpu.PrefetchScalarGridSpec(
        num_scalar_prefetch=2,
        grid=(NF, NB),
        in_specs=[
            pl.BlockSpec((BT, D), lambda f, b, be, nr: (b, 0)),
            pl.BlockSpec((1, FB, D), lambda f, b, be, nr: (be[b], f, 0)),
            pl.BlockSpec((1, FB, D), lambda f, b, be, nr: (be[b], f, 0)),
            pl.BlockSpec((1, D, FB), lambda f, b, be, nr: (be[b], 0, f)),
            pl.BlockSpec((BT, 1), lambda f, b, be, nr: (b, 0)),
        ],
        out_specs=pl.BlockSpec((RPAD, D), lambda f, b, be, nr: (0, 0)),
    )
    return pl.pallas_call(
        _mlp_body,
        grid_spec=grid_spec,
        out_shape=jax.ShapeDtypeStruct((RPAD, D), jnp.float32),
    )(block_expert, block_nrows, x_sorted, w_lin, w_v, w_1, sorted_w)


def _sc_combine_body(y_hbm, pos0_hbm, pos1_hbm, out_hbm,
                     i0, i1, r0, r1, rout, sem0, sem1):
    wid = lax.axis_index("s") * 2 + lax.axis_index("c")
    tokens_per = T // NW
    tch = tokens_per // 2
    for c in range(2):
        tbase = wid * tokens_per + c * tch
        pltpu.sync_copy(pos0_hbm.at[pl.ds(tbase, tch)], i0)
        pltpu.sync_copy(pos1_hbm.at[pl.ds(tbase, tch)], i1)
        d0 = pltpu.async_copy(y_hbm.at[i0], r0, sem0)
        d1 = pltpu.async_copy(y_hbm.at[i1], r1, sem1)
        d0.wait()
        d1.wait()

        @plsc.parallel_loop(0, tch, 1, unroll=2)
        def _(r):
            for j in range(D // 16):
                sl = pl.ds(j * 16, 16)
                rout[r, sl] = r0[r, sl] + r1[r, sl]

        pltpu.sync_copy(rout, out_hbm.at[pl.ds(tbase, tch)])


def _sc_combine(y_rows, pos0, pos1):
    tch = T // NW // 2
    mesh = plsc.VectorSubcoreMesh(core_axis_name="c", subcore_axis_name="s")
    k = functools.partial(
        pl.kernel,
        out_type=jax.ShapeDtypeStruct((T, D), jnp.float32),
        mesh=mesh,
        scratch_types=[
            pltpu.VMEM((tch,), jnp.int32),
            pltpu.VMEM((tch,), jnp.int32),
            pltpu.VMEM((tch, D), jnp.float32),
            pltpu.VMEM((tch, D), jnp.float32),
            pltpu.VMEM((tch, D), jnp.float32),
            pltpu.SemaphoreType.DMA,
            pltpu.SemaphoreType.DMA,
        ],
    )(_sc_combine_body)
    return k(y_rows, pos0, pos1)


def kernel(hidden_states, gate_w, w_lin, w_v, w_1):
    x = hidden_states.reshape(T, D)
    logits, rw, sel = _router(x, gate_w)

    flat_e = sel.reshape(-1)
    oh = (flat_e[:, None] == jnp.arange(E, dtype=jnp.int32)[None, :])
    oh = oh.astype(jnp.int32)
    counts = jnp.sum(oh, axis=0)
    ranks = jnp.sum((jnp.cumsum(oh, axis=0) - oh) * oh, axis=1)
    nblk = (counts + BT - 1) // BT
    blk_end = jnp.cumsum(nblk)
    padded_start = (blk_end - nblk) * BT
    pos = jnp.sum(padded_start[None, :] * oh, axis=1) + ranks
    sorted_w = jnp.zeros((RPAD,), jnp.float32).at[pos].set(
        rw.reshape(-1)).reshape(RPAD, 1)
    bids = jnp.arange(NB, dtype=jnp.int32)
    block_expert = jnp.searchsorted(blk_end, bids, side="right")
    block_expert = jnp.minimum(block_expert, E - 1).astype(jnp.int32)
    blk_start = blk_end - nblk
    block_nrows = jnp.clip(
        counts[block_expert] - (bids - blk_start[block_expert]) * BT,
        0, BT).astype(jnp.int32)
    pos_r = pos.reshape(T, K).astype(jnp.int32)
    pos0 = pos_r[:, 0]
    pos1 = pos_r[:, 1]
    posb = jnp.stack([pos0.reshape(NW, _TPW), pos1.reshape(NW, _TPW)],
                     axis=1)

    x_sorted = _sc_dispatch(x, posb)
    y_rows = _mlp(x_sorted, w_lin, w_v, w_1, sorted_w, block_expert,
                  block_nrows)
    final = _sc_combine(y_rows, pos0, pos1)
    return final.reshape(1, T, D), logits


# speedup vs baseline: 2.7325x; 1.1795x over previous
import functools
import math

import jax
import jax.numpy as jnp
from jax import lax
from jax.experimental import pallas as pl
from jax.experimental.pallas import tpu as pltpu
from jax.experimental.pallas import tpu_sc as plsc

D = 1024
F = 2048
E = 8
K = 2
T = 2048
RT = T * K
BT = 256
FB = 1024
NF = F // FB
NB = RT // BT + E
RPAD = NB * BT
NW = 32


def _router_body(x_ref, gw_ref, logits_ref, rw_ref, sel_ref):
    x = x_ref[...]
    logits = lax.dot_general(x, gw_ref[...], (((1,), (1,)), ((), ())),
                             preferred_element_type=jnp.float32)
    logits_ref[...] = logits
    m = jnp.max(logits, axis=1, keepdims=True)
    ex = jnp.exp(logits - m)
    p = ex / jnp.sum(ex, axis=1, keepdims=True)
    iota = lax.broadcasted_iota(jnp.int32, (T, E), 1)
    m1 = jnp.max(p, axis=1, keepdims=True)
    i1 = jnp.min(jnp.where(p == m1, iota, E), axis=1, keepdims=True)
    p2 = jnp.where(iota == i1, -1.0, p)
    m2 = jnp.max(p2, axis=1, keepdims=True)
    i2 = jnp.min(jnp.where(p2 == m2, iota, E), axis=1, keepdims=True)
    rw_ref[...] = jnp.concatenate([m1, m2], axis=1)
    sel_ref[...] = jnp.concatenate([i1, i2], axis=1)


def _router(x, gate_w):
    return pl.pallas_call(
        _router_body,
        out_shape=(
            jax.ShapeDtypeStruct((T, E), jnp.float32),
            jax.ShapeDtypeStruct((T, K), jnp.float32),
            jax.ShapeDtypeStruct((T, K), jnp.int32),
        ),
    )(x, gate_w)


_TPW = T // NW


def _sc_dispatch_body(x_hbm, posb_hbm, out_hbm, xbuf, i0, i1, sem0, sem1):
    wid = lax.axis_index("s") * 2 + lax.axis_index("c")
    pltpu.sync_copy(x_hbm.at[pl.ds(wid * _TPW, _TPW)], xbuf)
    pltpu.sync_copy(posb_hbm.at[wid, 0], i0)
    pltpu.sync_copy(posb_hbm.at[wid, 1], i1)
    d0 = pltpu.async_copy(xbuf, out_hbm.at[i0], sem0)
    d1 = pltpu.async_copy(xbuf, out_hbm.at[i1], sem1)
    d0.wait()
    d1.wait()


def _sc_dispatch(x, posb):
    mesh = plsc.VectorSubcoreMesh(core_axis_name="c", subcore_axis_name="s")
    k = functools.partial(
        pl.kernel,
        out_type=jax.ShapeDtypeStruct((RPAD, D), jnp.float32),
        mesh=mesh,
        scratch_types=[
            pltpu.VMEM((_TPW, D), jnp.float32),
            pltpu.VMEM((_TPW,), jnp.int32),
            pltpu.VMEM((_TPW,), jnp.int32),
            pltpu.SemaphoreType.DMA,
            pltpu.SemaphoreType.DMA,
        ],
    )(_sc_dispatch_body)
    return k(x, posb)


_INV_SQRT2 = 1.0 / math.sqrt(2.0)


def _mlp_body(be_ref, nr_ref, x_ref, wl_ref, wv_ref, w1_ref, w_ref, y_ref):
    f = pl.program_id(0)
    b = pl.program_id(1)

    @pl.when(nr_ref[b] > 0)
    def _():
        xb = x_ref[...]
        a = lax.dot_general(xb, wl_ref[0], (((1,), (1,)), ((), ())),
                            preferred_element_type=jnp.float32)
        v = lax.dot_general(xb, wv_ref[0], (((1,), (1,)), ((), ())),
                            preferred_element_type=jnp.float32)
        g = 0.5 * a * (1.0 + lax.erf(a * _INV_SQRT2))
        p = lax.dot_general(g * v, w1_ref[0], (((1,), (1,)), ((), ())),
                            preferred_element_type=jnp.float32)
        row = pl.ds(b * BT, BT)

        @pl.when(f == 0)
        def _():
            y_ref[row, :] = p

        @pl.when(jnp.logical_and(f > 0, f < NF - 1))
        def _():
            y_ref[row, :] += p

        @pl.when(f == NF - 1)
        def _():
            y_ref[row, :] = (y_ref[row, :] + p) * w_ref[...]


def _mlp(x_sorted, w_lin, w_v, w_1, sorted_w, block_expert, block_nrows):
    grid_spec = pltpu.PrefetchScalarGridSpec(
        num_scalar_prefetch=2,
        grid=(NF, NB),
        in_specs=[
            pl.BlockSpec((BT, D), lambda f, b, be, nr: (b, 0)),
            pl.BlockSpec((1, FB, D), lambda f, b, be, nr: (be[b], f, 0)),
            pl.BlockSpec((1, FB, D), lambda f, b, be, nr: (be[b], f, 0)),
            pl.BlockSpec((1, D, FB), lambda f, b, be, nr: (be[b], 0, f)),
            pl.BlockSpec((BT, 1), lambda f, b, be, nr: (b, 0)),
        ],
        out_specs=pl.BlockSpec((RPAD, D), lambda f, b, be, nr: (0, 0)),
    )
    return pl.pallas_call(
        _mlp_body,
        grid_spec=grid_spec,
        out_shape=jax.ShapeDtypeStruct((RPAD, D), jnp.float32),
    )(block_expert, block_nrows, x_sorted, w_lin, w_v, w_1, sorted_w)


def _sc_combine_body(y_hbm, pos0_hbm, pos1_hbm, out_hbm,
                     i0, i1, r0, r1, rout, sem0, sem1):
    wid = lax.axis_index("s") * 2 + lax.axis_index("c")
    tokens_per = T // NW
    tch = tokens_per // 2
    for c in range(2):
        tbase = wid * tokens_per + c * tch
        pltpu.sync_copy(pos0_hbm.at[pl.ds(tbase, tch)], i0)
        pltpu.sync_copy(pos1_hbm.at[pl.ds(tbase, tch)], i1)
        d0 = pltpu.async_copy(y_hbm.at[i0], r0, sem0)
        d1 = pltpu.async_copy(y_hbm.at[i1], r1, sem1)
        d0.wait()
        d1.wait()

        @plsc.parallel_loop(0, tch, 1, unroll=2)
        def _(r):
            for j in range(D // 16):
                sl = pl.ds(j * 16, 16)
                rout[r, sl] = r0[r, sl] + r1[r, sl]

        pltpu.sync_copy(rout, out_hbm.at[pl.ds(tbase, tch)])


def _sc_combine(y_rows, pos0, pos1):
    tch = T // NW // 2
    mesh = plsc.VectorSubcoreMesh(core_axis_name="c", subcore_axis_name="s")
    k = functools.partial(
        pl.kernel,
        out_type=jax.ShapeDtypeStruct((T, D), jnp.float32),
        mesh=mesh,
        scratch_types=[
            pltpu.VMEM((tch,), jnp.int32),
            pltpu.VMEM((tch,), jnp.int32),
            pltpu.VMEM((tch, D), jnp.float32),
            pltpu.VMEM((tch, D), jnp.float32),
            pltpu.VMEM((tch, D), jnp.float32),
            pltpu.SemaphoreType.DMA,
            pltpu.SemaphoreType.DMA,
        ],
    )(_sc_combine_body)
    return k(y_rows, pos0, pos1)


def kernel(hidden_states, gate_w, w_lin, w_v, w_1):
    x = hidden_states.reshape(T, D)
    logits, rw, sel = _router(x, gate_w)

    flat_e = sel.reshape(-1)
    oh = (flat_e[:, None] == jnp.arange(E, dtype=jnp.int32)[None, :])
    oh = oh.astype(jnp.int32)
    counts = jnp.sum(oh, axis=0)
    ranks = jnp.sum((jnp.cumsum(oh, axis=0) - oh) * oh, axis=1)
    nblk = (counts + BT - 1) // BT
    blk_end = jnp.cumsum(nblk)
    padded_start = (blk_end - nblk) * BT
    pos = jnp.sum(padded_start[None, :] * oh, axis=1) + ranks
    sorted_w = jnp.zeros((RPAD,), jnp.float32).at[pos].set(
        rw.reshape(-1)).reshape(RPAD, 1)
    bids = jnp.arange(NB, dtype=jnp.int32)
    block_expert = jnp.searchsorted(blk_end, bids, side="right")
    block_expert = jnp.minimum(block_expert, E - 1).astype(jnp.int32)
    blk_start = blk_end - nblk
    block_nrows = jnp.clip(
        counts[block_expert] - (bids - blk_start[block_expert]) * BT,
        0, BT).astype(jnp.int32)
    pos_r = pos.reshape(T, K).astype(jnp.int32)
    pos0 = pos_r[:, 0]
    pos1 = pos_r[:, 1]
    posb = jnp.stack([pos0.reshape(NW, _TPW), pos1.reshape(NW, _TPW)],
                     axis=1)

    x_sorted = _sc_dispatch(x, posb)
    y_rows = _mlp(x_sorted, w_lin, w_v, w_1, sorted_w, block_expert,
                  block_nrows)
    final = _sc_combine(y_rows, pos0, pos1)
    return final.reshape(1, T, D), logits
